# Initial kernel scaffold; baseline (speedup 1.0000x reference)
import jax
import jax.numpy as jnp
from jax.experimental import pallas as pl


def kernel(x, edge_index, edge_weight, W_first, b_first, W_conv, b_conv, W_lin2, b_lin2):
    raise NotImplementedError("write your pallas kernel here")


# speedup vs baseline: 13.6509x; 13.6509x over previous
import jax
import jax.numpy as jnp
from jax import lax
from jax.experimental import pallas as pl
from jax.experimental.pallas import tpu as pltpu
from jax.experimental.pallas import tpu_sc as plsc

NC = 2
NS = 16
NW = NC * NS
HID = 16


def _sc_mesh():
    return plsc.VectorSubcoreMesh(
        core_axis_name="c", subcore_axis_name="s", num_cores=NC, num_subcores=NS
    )


def _deg_call(col2d, w2d, NP):
    RE = col2d.shape[0]
    rows_per_w = RE // NW
    CH = 16
    n_chunks = rows_per_w // CH
    npw = NP // NS

    def body(col_hbm, w_hbm, out_hbm, colv, wv, zbuf, degsh):
        c = lax.axis_index("c")
        s = lax.axis_index("s")
        wid = c * NS + s
        zeros = jnp.zeros((16,), jnp.float32)

        def zb(i, _):
            zbuf[pl.ds(i * 16, 16)] = zeros
            return 0

        lax.fori_loop(0, npw // 16, zb, 0)
        pltpu.sync_copy(zbuf, degsh.at[pl.ds(s * npw, npw)])
        plsc.subcore_barrier()

        def chunk(k, _):
            base = wid * rows_per_w + k * CH
            pltpu.sync_copy(col_hbm.at[pl.ds(base, CH)], colv)
            pltpu.sync_copy(w_hbm.at[pl.ds(base, CH)], wv)
            for r in range(CH):
                pltpu.sync_copy(wv.at[r], degsh.at[colv.at[r]], add=True)
            return 0

        lax.fori_loop(0, n_chunks, chunk, 0)
        plsc.subcore_barrier()
        pltpu.sync_copy(degsh.at[pl.ds(s * npw, npw)], out_hbm.at[c, pl.ds(s * npw, npw)])

    return pl.kernel(
        body,
        out_type=jax.ShapeDtypeStruct((NC, NP), jnp.float32),
        mesh=_sc_mesh(),
        scratch_types=[
            pltpu.VMEM((CH, 128), jnp.int32),
            pltpu.VMEM((CH, 128), jnp.float32),
            pltpu.VMEM((npw,), jnp.float32),
            pltpu.MemorySpace.VMEM_SHARED((NP,), jnp.float32),
        ],
    )(col2d, w2d)


def _msg_call(row2d, col2d, w2d, tab, NP):
    RE = row2d.shape[0]
    rows_per_w = RE // NW
    n_chunks = rows_per_w
    NF = NP * HID
    npw = NF // NS
    ZR = 1024

    def body(row_hbm, col_hbm, w_hbm, tab_hbm, out_hbm,
             rowv, colv, wv, rbuf, mbuf, ibuf, zbuf, acc, sem):
        IOTA = lax.iota(jnp.int32, 16)
        c = lax.axis_index("c")
        s = lax.axis_index("s")
        wid = c * NS + s
        zeros = jnp.zeros((16,), jnp.float32)

        def zb(i, _):
            zbuf[pl.ds(i * 16, 16)] = zeros
            return 0

        lax.fori_loop(0, ZR // 16, zb, 0)

        def zc(i, _):
            pltpu.sync_copy(zbuf, acc.at[pl.ds(s * npw + i * ZR, ZR)])
            return 0

        lax.fori_loop(0, npw // ZR, zc, 0)
        plsc.subcore_barrier()

        def chunk(k, _):
            base = wid * rows_per_w + k
            pltpu.sync_copy(w_hbm.at[base], wv)
            for h in range(2):
                pltpu.sync_copy(row_hbm.at[base, pl.ds(h * 64, 64)], rowv)
                pltpu.sync_copy(col_hbm.at[base, pl.ds(h * 64, 64)], colv)
                pltpu.async_copy(tab_hbm.at[rowv], rbuf, sem).wait()
                for q in range(4):
                    w16 = wv[pl.ds(h * 64 + q * 16, 16)]
                    c16 = colv[pl.ds(q * 16, 16)]
                    for j in range(16):
                        e = q * 16 + j
                        mbuf[e >> 3, pl.ds((e & 7) * 16, 16)] = (
                            rbuf[e, pl.ds(0, 16)] * w16[j]
                        )
                        ibuf[e >> 3, pl.ds((e & 7) * 16, 16)] = c16[j] * HID + IOTA
                for r in range(8):
                    pltpu.sync_copy(mbuf.at[r], acc.at[ibuf.at[r]], add=True)
            return 0

        lax.fori_loop(0, n_chunks, chunk, 0)
        plsc.subcore_barrier()
        pltpu.sync_copy(acc.at[pl.ds(s * npw, npw)], out_hbm.at[c, pl.ds(s * npw, npw)])

    return pl.kernel(
        body,
        out_type=jax.ShapeDtypeStruct((NC, NF), jnp.float32),
        mesh=_sc_mesh(),
        scratch_types=[
            pltpu.VMEM((64,), jnp.int32),
            pltpu.VMEM((64,), jnp.int32),
            pltpu.VMEM((128,), jnp.float32),
            pltpu.VMEM((64, 128), jnp.float32),
            pltpu.VMEM((8, 128), jnp.float32),
            pltpu.VMEM((8, 128), jnp.int32),
            pltpu.VMEM((ZR,), jnp.float32),
            pltpu.MemorySpace.VMEM_SHARED((NF,), jnp.float32),
            pltpu.SemaphoreType.DMA,
        ],
    )(row2d, col2d, w2d, tab)


def _dis_call(degp3):
    NR = degp3.shape[1]
    BR = 784

    def body(d_ref, o_ref):
        deg = d_ref[0] + d_ref[1] + 1.0
        o_ref[...] = jnp.where(deg > 0, lax.rsqrt(deg), 0.0)

    return pl.pallas_call(
        body,
        grid=(NR // BR,),
        in_specs=[pl.BlockSpec((2, BR, 16), lambda i: (0, i, 0))],
        out_specs=pl.BlockSpec((BR, 16), lambda i: (i, 0)),
        out_shape=jax.ShapeDtypeStruct((NR, 16), jnp.float32),
    )(degp3)


def _dense_call(x, Wf, bf, Wc, dis16):
    NN, F = x.shape
    BR = 800

    def body(x_ref, wf_ref, bf_ref, wc_ref, d_ref, o_ref, o128_ref):
        h = lax.dot_general(
            x_ref[...], wf_ref[...], (((1,), (1,)), ((), ())),
            preferred_element_type=jnp.float32,
        )
        h = jnp.maximum(h + bf_ref[...], 0.0)
        h2 = lax.dot_general(
            h, wc_ref[...], (((1,), (1,)), ((), ())),
            preferred_element_type=jnp.float32,
        )
        t = h2 * d_ref[...]
        o_ref[...] = t
        o128_ref[...] = jnp.concatenate([t] * 8, axis=1)

    return pl.pallas_call(
        body,
        grid=(NN // BR,),
        in_specs=[
            pl.BlockSpec((BR, F), lambda i: (i, 0)),
            pl.BlockSpec((HID, F), lambda i: (0, 0)),
            pl.BlockSpec((1, HID), lambda i: (0, 0)),
            pl.BlockSpec((HID, HID), lambda i: (0, 0)),
            pl.BlockSpec((BR, HID), lambda i: (i, 0)),
        ],
        out_specs=[
            pl.BlockSpec((BR, HID), lambda i: (i, 0)),
            pl.BlockSpec((BR, 128), lambda i: (i, 0)),
        ],
        out_shape=[
            jax.ShapeDtypeStruct((NN, HID), jnp.float32),
            jax.ShapeDtypeStruct((NN, 128), jnp.float32),
        ],
    )(x, Wf, bf.reshape(1, HID), Wc, dis16)


def _final_call(accp, h2s, dis16, bc, Wl, bl):
    NN = h2s.shape[0]
    C = Wl.shape[0]
    BR = 800

    def body(a_ref, h_ref, d_ref, bc_ref, wl_ref, bl_ref, o_ref):
        m = a_ref[0] + a_ref[1] + h_ref[...]
        h = jnp.maximum(d_ref[...] * m + bc_ref[...], 0.0)
        logits = lax.dot_general(
            h, wl_ref[...], (((1,), (1,)), ((), ())),
            preferred_element_type=jnp.float32,
        ) + bl_ref[...]
        mx = jnp.max(logits, axis=-1, keepdims=True)
        ex = jnp.exp(logits - mx)
        lse = mx + jnp.log(jnp.sum(ex, axis=-1, keepdims=True))
        o_ref[...] = logits - lse

    return pl.pallas_call(
        body,
        grid=(NN // BR,),
        in_specs=[
            pl.BlockSpec((2, BR, HID), lambda i: (0, i, 0)),
            pl.BlockSpec((BR, HID), lambda i: (i, 0)),
            pl.BlockSpec((BR, HID), lambda i: (i, 0)),
            pl.BlockSpec((1, HID), lambda i: (0, 0)),
            pl.BlockSpec((C, HID), lambda i: (0, 0)),
            pl.BlockSpec((1, C), lambda i: (0, 0)),
        ],
        out_specs=pl.BlockSpec((BR, C), lambda i: (i, 0)),
        out_shape=jax.ShapeDtypeStruct((NN, C), jnp.float32),
    )(accp, h2s, dis16, bc.reshape(1, HID), Wl, bl.reshape(1, C))


def kernel(x, edge_index, edge_weight, W_first, b_first, W_conv, b_conv, W_lin2, b_lin2):
    NN, _ = x.shape
    E = edge_index.shape[1]

    NP = ((NN + 2047) // 2048) * 2048
    base_rows = -(-E // 128)
    RE = -(-base_rows // (NW * 16)) * (NW * 16)
    padE = RE * 128 - E

    row = edge_index[0].astype(jnp.int32)
    col = edge_index[1].astype(jnp.int32)
    w = edge_weight.astype(jnp.float32)
    ar = jnp.arange(padE, dtype=jnp.int32)
    pad_dst = NN + ar % (NP - NN) if NP > NN else ar % NN
    row2d = jnp.concatenate([row, ar % NN]).reshape(RE, 128)
    col2d = jnp.concatenate([col, pad_dst]).reshape(RE, 128)
    w2d = jnp.concatenate([w, jnp.zeros((padE,), jnp.float32)]).reshape(RE, 128)

    degp = _deg_call(col2d, w2d, NP)
    dis2d = _dis_call(degp.reshape(2, NP // 16, 16))
    dis16 = jnp.broadcast_to(dis2d.reshape(NP)[:NN, None], (NN, HID))
    h2s, tab128 = _dense_call(x, W_first, b_first, W_conv, dis16)
    accp = _msg_call(row2d, col2d, w2d, tab128, NP).reshape(2, NP, HID)
    return _final_call(accp, h2s, dis16, b_conv, W_lin2, b_lin2)


# speedup vs baseline: 28.6524x; 2.0989x over previous
import jax
import jax.numpy as jnp
from jax import lax
from jax.experimental import pallas as pl
from jax.experimental.pallas import tpu as pltpu
from jax.experimental.pallas import tpu_sc as plsc

NC = 2
NS = 16
NW = NC * NS
HID = 16


def _sc_mesh():
    return plsc.VectorSubcoreMesh(
        core_axis_name="c", subcore_axis_name="s", num_cores=NC, num_subcores=NS
    )


def _deg_call(col2d, w2d, NP):
    RE = col2d.shape[0]
    rows_per_w = RE // NW
    CH = 16
    n_chunks = rows_per_w // CH
    npw = NP // NS

    def body(col_hbm, w_hbm, out_hbm, colv, wv, zbuf, degsh):
        c = lax.axis_index("c")
        s = lax.axis_index("s")
        wid = c * NS + s
        zeros = jnp.zeros((16,), jnp.float32)

        def zb(i, _):
            zbuf[pl.ds(i * 16, 16)] = zeros
            return 0

        lax.fori_loop(0, npw // 16, zb, 0)
        pltpu.sync_copy(zbuf, degsh.at[pl.ds(s * npw, npw)])
        plsc.subcore_barrier()

        def chunk(k, _):
            base = wid * rows_per_w + k * CH
            pltpu.sync_copy(col_hbm.at[pl.ds(base, CH)], colv)
            pltpu.sync_copy(w_hbm.at[pl.ds(base, CH)], wv)
            for r in range(CH):
                pltpu.sync_copy(wv.at[r], degsh.at[colv.at[r]], add=True)
            return 0

        lax.fori_loop(0, n_chunks, chunk, 0)
        plsc.subcore_barrier()
        pltpu.sync_copy(degsh.at[pl.ds(s * npw, npw)], out_hbm.at[c, pl.ds(s * npw, npw)])

    return pl.kernel(
        body,
        out_type=jax.ShapeDtypeStruct((NC, NP), jnp.float32),
        mesh=_sc_mesh(),
        scratch_types=[
            pltpu.VMEM((CH, 128), jnp.int32),
            pltpu.VMEM((CH, 128), jnp.float32),
            pltpu.VMEM((npw,), jnp.float32),
            pltpu.MemorySpace.VMEM_SHARED((NP,), jnp.float32),
        ],
    )(col2d, w2d)


def _msg_call(row2d, col2d, w2d, tab, NP):
    RE = row2d.shape[0]
    rows_per_w = RE // NW
    BK = 4
    n_blocks = rows_per_w // BK
    NG = BK * 4
    NF = NP * HID
    npw = NF // NS
    ZR = 1024

    def body(row_hbm, col_hbm, w_hbm, tab_hbm, out_hbm,
             rowB, colB, wB, rbuf, mbuf, ibuf, zbuf, acc, semg, sems):
        IOTA = lax.iota(jnp.int32, 16)
        c = lax.axis_index("c")
        s = lax.axis_index("s")
        wid = c * NS + s
        zeros = jnp.zeros((16,), jnp.float32)

        def zb(i, _):
            zbuf[pl.ds(i * 16, 16)] = zeros
            return 0

        lax.fori_loop(0, ZR // 16, zb, 0)

        def zc(i, _):
            pltpu.sync_copy(zbuf, acc.at[pl.ds(s * npw + i * ZR, ZR)])
            return 0

        lax.fori_loop(0, npw // ZR, zc, 0)
        plsc.subcore_barrier()

        def gather(g, slot):
            r, h = g >> 2, g & 3
            return pltpu.async_copy(
                tab_hbm.at[rowB.at[r, pl.ds(h * 32, 32)]], rbuf.at[slot], semg
            )

        def block(k, _):
            base = wid * n_blocks + k
            pltpu.sync_copy(row_hbm.at[pl.ds(base * BK, BK)], rowB)
            pltpu.sync_copy(col_hbm.at[pl.ds(base * BK, BK)], colB)
            pltpu.sync_copy(w_hbm.at[pl.ds(base * BK, BK)], wB)
            gd = [None, None]
            sd = [[], []]
            gd[0] = gather(0, 0)
            for g in range(NG):
                slot = g & 1
                if g + 1 < NG:
                    gd[1 - slot] = gather(g + 1, 1 - slot)
                gd[slot].wait()
                for d in sd[slot]:
                    d.wait()
                r, h = g >> 2, g & 3
                for q in range(2):
                    w16 = wB[r, pl.ds(h * 32 + q * 16, 16)]
                    c16 = colB[r, pl.ds(h * 32 + q * 16, 16)]
                    for j in range(16):
                        e = q * 16 + j
                        mbuf[slot, e >> 3, pl.ds((e & 7) * 16, 16)] = (
                            rbuf[slot, e, pl.ds(0, 16)] * w16[j]
                        )
                        ibuf[slot, e >> 3, pl.ds((e & 7) * 16, 16)] = (
                            c16[j] * HID + IOTA
                        )
                sd[slot] = [
                    pltpu.async_copy(
                        mbuf.at[slot, q], acc.at[ibuf.at[slot, q]], sems, add=True
                    )
                    for q in range(4)
                ]
            for slot in range(2):
                for d in sd[slot]:
                    d.wait()
            return 0

        lax.fori_loop(0, n_blocks, block, 0)
        plsc.subcore_barrier()
        pltpu.sync_copy(acc.at[pl.ds(s * npw, npw)], out_hbm.at[c, pl.ds(s * npw, npw)])

    return pl.kernel(
        body,
        out_type=jax.ShapeDtypeStruct((NC, NF), jnp.float32),
        mesh=_sc_mesh(),
        scratch_types=[
            pltpu.VMEM((BK, 128), jnp.int32),
            pltpu.VMEM((BK, 128), jnp.int32),
            pltpu.VMEM((BK, 128), jnp.float32),
            pltpu.VMEM((2, 32, 128), jnp.float32),
            pltpu.VMEM((2, 4, 128), jnp.float32),
            pltpu.VMEM((2, 4, 128), jnp.int32),
            pltpu.VMEM((ZR,), jnp.float32),
            pltpu.MemorySpace.VMEM_SHARED((NF,), jnp.float32),
            pltpu.SemaphoreType.DMA,
            pltpu.SemaphoreType.DMA,
        ],
    )(row2d, col2d, w2d, tab)


def _dis_call(degp3):
    NR = degp3.shape[1]
    BR = 784

    def body(d_ref, o_ref):
        deg = d_ref[0] + d_ref[1] + 1.0
        o_ref[...] = jnp.where(deg > 0, lax.rsqrt(deg), 0.0)

    return pl.pallas_call(
        body,
        grid=(NR // BR,),
        in_specs=[pl.BlockSpec((2, BR, 16), lambda i: (0, i, 0))],
        out_specs=pl.BlockSpec((BR, 16), lambda i: (i, 0)),
        out_shape=jax.ShapeDtypeStruct((NR, 16), jnp.float32),
    )(degp3)


def _dense_call(x, Wf, bf, Wc, dis16):
    NN, F = x.shape
    BR = 800

    def body(x_ref, wf_ref, bf_ref, wc_ref, d_ref, o_ref, o128_ref):
        h = lax.dot_general(
            x_ref[...], wf_ref[...], (((1,), (1,)), ((), ())),
            preferred_element_type=jnp.float32,
        )
        h = jnp.maximum(h + bf_ref[...], 0.0)
        h2 = lax.dot_general(
            h, wc_ref[...], (((1,), (1,)), ((), ())),
            preferred_element_type=jnp.float32,
        )
        t = h2 * d_ref[...]
        o_ref[...] = t
        o128_ref[...] = jnp.concatenate([t] * 8, axis=1)

    return pl.pallas_call(
        body,
        grid=(NN // BR,),
        in_specs=[
            pl.BlockSpec((BR, F), lambda i: (i, 0)),
            pl.BlockSpec((HID, F), lambda i: (0, 0)),
            pl.BlockSpec((1, HID), lambda i: (0, 0)),
            pl.BlockSpec((HID, HID), lambda i: (0, 0)),
            pl.BlockSpec((BR, HID), lambda i: (i, 0)),
        ],
        out_specs=[
            pl.BlockSpec((BR, HID), lambda i: (i, 0)),
            pl.BlockSpec((BR, 128), lambda i: (i, 0)),
        ],
        out_shape=[
            jax.ShapeDtypeStruct((NN, HID), jnp.float32),
            jax.ShapeDtypeStruct((NN, 128), jnp.float32),
        ],
    )(x, Wf, bf.reshape(1, HID), Wc, dis16)


def _final_call(accp, h2s, dis16, bc, Wl, bl):
    NN = h2s.shape[0]
    C = Wl.shape[0]
    BR = 800

    def body(a_ref, h_ref, d_ref, bc_ref, wl_ref, bl_ref, o_ref):
        m = a_ref[0] + a_ref[1] + h_ref[...]
        h = jnp.maximum(d_ref[...] * m + bc_ref[...], 0.0)
        logits = lax.dot_general(
            h, wl_ref[...], (((1,), (1,)), ((), ())),
            preferred_element_type=jnp.float32,
        ) + bl_ref[...]
        mx = jnp.max(logits, axis=-1, keepdims=True)
        ex = jnp.exp(logits - mx)
        lse = mx + jnp.log(jnp.sum(ex, axis=-1, keepdims=True))
        o_ref[...] = logits - lse

    return pl.pallas_call(
        body,
        grid=(NN // BR,),
        in_specs=[
            pl.BlockSpec((2, BR, HID), lambda i: (0, i, 0)),
            pl.BlockSpec((BR, HID), lambda i: (i, 0)),
            pl.BlockSpec((BR, HID), lambda i: (i, 0)),
            pl.BlockSpec((1, HID), lambda i: (0, 0)),
            pl.BlockSpec((C, HID), lambda i: (0, 0)),
            pl.BlockSpec((1, C), lambda i: (0, 0)),
        ],
        out_specs=pl.BlockSpec((BR, C), lambda i: (i, 0)),
        out_shape=jax.ShapeDtypeStruct((NN, C), jnp.float32),
    )(accp, h2s, dis16, bc.reshape(1, HID), Wl, bl.reshape(1, C))


def kernel(x, edge_index, edge_weight, W_first, b_first, W_conv, b_conv, W_lin2, b_lin2):
    NN, _ = x.shape
    E = edge_index.shape[1]

    NP = ((NN + 2047) // 2048) * 2048
    base_rows = -(-E // 128)
    RE = -(-base_rows // (NW * 16)) * (NW * 16)
    padE = RE * 128 - E

    row = edge_index[0].astype(jnp.int32)
    col = edge_index[1].astype(jnp.int32)
    w = edge_weight.astype(jnp.float32)
    ar = jnp.arange(padE, dtype=jnp.int32)
    pad_dst = NN + ar % (NP - NN) if NP > NN else ar % NN
    row2d = jnp.concatenate([row, ar % NN]).reshape(RE, 128)
    col2d = jnp.concatenate([col, pad_dst]).reshape(RE, 128)
    w2d = jnp.concatenate([w, jnp.zeros((padE,), jnp.float32)]).reshape(RE, 128)

    degp = _deg_call(col2d, w2d, NP)
    dis2d = _dis_call(degp.reshape(2, NP // 16, 16))
    dis16 = jnp.broadcast_to(dis2d.reshape(NP)[:NN, None], (NN, HID))
    h2s, tab128 = _dense_call(x, W_first, b_first, W_conv, dis16)
    accp = _msg_call(row2d, col2d, w2d, tab128, NP).reshape(2, NP, HID)
    return _final_call(accp, h2s, dis16, b_conv, W_lin2, b_lin2)


# speedup vs baseline: 30.8717x; 1.0775x over previous
import jax
import jax.numpy as jnp
from jax import lax
from jax.experimental import pallas as pl
from jax.experimental.pallas import tpu as pltpu
from jax.experimental.pallas import tpu_sc as plsc

NC = 2
NS = 16
NW = NC * NS
HID = 16


def _sc_mesh():
    return plsc.VectorSubcoreMesh(
        core_axis_name="c", subcore_axis_name="s", num_cores=NC, num_subcores=NS
    )


def _deg_call(col2d, w2d, NP):
    RE = col2d.shape[0]
    rows_per_w = RE // NW
    CH = 16
    n_chunks = rows_per_w // CH
    npw = NP // NS

    def body(col_hbm, w_hbm, out_hbm, colv, wv, zbuf, degsh):
        c = lax.axis_index("c")
        s = lax.axis_index("s")
        wid = c * NS + s
        zeros = jnp.zeros((16,), jnp.float32)

        def zb(i, _):
            zbuf[pl.ds(i * 16, 16)] = zeros
            return 0

        lax.fori_loop(0, npw // 16, zb, 0)
        pltpu.sync_copy(zbuf, degsh.at[pl.ds(s * npw, npw)])
        plsc.subcore_barrier()

        def chunk(k, _):
            base = wid * rows_per_w + k * CH
            pltpu.sync_copy(col_hbm.at[pl.ds(base, CH)], colv)
            pltpu.sync_copy(w_hbm.at[pl.ds(base, CH)], wv)
            for r in range(CH):
                pltpu.sync_copy(wv.at[r], degsh.at[colv.at[r]], add=True)
            return 0

        lax.fori_loop(0, n_chunks, chunk, 0)
        plsc.subcore_barrier()
        pltpu.sync_copy(degsh.at[pl.ds(s * npw, npw)], out_hbm.at[c, pl.ds(s * npw, npw)])

    return pl.kernel(
        body,
        out_type=jax.ShapeDtypeStruct((NC, NP), jnp.float32),
        mesh=_sc_mesh(),
        scratch_types=[
            pltpu.VMEM((CH, 128), jnp.int32),
            pltpu.VMEM((CH, 128), jnp.float32),
            pltpu.VMEM((npw,), jnp.float32),
            pltpu.MemorySpace.VMEM_SHARED((NP,), jnp.float32),
        ],
    )(col2d, w2d)


def _msg_call(row2d, col2d, w2d, tab, NP):
    RE = row2d.shape[0]
    rows_per_w = RE // NW
    BK = 8
    n_blocks = rows_per_w // BK
    NG = BK * 4
    NF = NP * HID
    npw = NF // NS
    ZR = 1024

    def body(row_hbm, col_hbm, w_hbm, tab_hbm, out_hbm,
             rowB, colB, wB, rbuf, mbuf, ibuf, zbuf, acc, semg, sems):
        IOTA = lax.iota(jnp.int32, 16)
        c = lax.axis_index("c")
        s = lax.axis_index("s")
        wid = c * NS + s
        zeros = jnp.zeros((16,), jnp.float32)

        def zb(i, _):
            zbuf[pl.ds(i * 16, 16)] = zeros
            return 0

        lax.fori_loop(0, ZR // 16, zb, 0)

        def zc(i, _):
            pltpu.sync_copy(zbuf, acc.at[pl.ds(s * npw + i * ZR, ZR)])
            return 0

        lax.fori_loop(0, npw // ZR, zc, 0)
        plsc.subcore_barrier()

        def gather(g, slot):
            r, h = g >> 2, g & 3
            return pltpu.async_copy(
                tab_hbm.at[rowB.at[r, pl.ds(h * 32, 32)]], rbuf.at[slot], semg
            )

        def block(k, _):
            base = wid * n_blocks + k
            pltpu.sync_copy(row_hbm.at[pl.ds(base * BK, BK)], rowB)
            pltpu.sync_copy(col_hbm.at[pl.ds(base * BK, BK)], colB)
            pltpu.sync_copy(w_hbm.at[pl.ds(base * BK, BK)], wB)
            gd = [None, None]
            sd = [[], []]
            gd[0] = gather(0, 0)
            for g in range(NG):
                slot = g & 1
                if g + 1 < NG:
                    gd[1 - slot] = gather(g + 1, 1 - slot)
                gd[slot].wait()
                for d in sd[slot]:
                    d.wait()
                r, h = g >> 2, g & 3
                for q in range(2):
                    w16 = wB[r, pl.ds(h * 32 + q * 16, 16)]
                    c16 = colB[r, pl.ds(h * 32 + q * 16, 16)]
                    for j in range(16):
                        e = q * 16 + j
                        mbuf[slot, e >> 3, pl.ds((e & 7) * 16, 16)] = (
                            rbuf[slot, e, pl.ds(0, 16)] * w16[j]
                        )
                        ibuf[slot, e >> 3, pl.ds((e & 7) * 16, 16)] = (
                            c16[j] * HID + IOTA
                        )
                sd[slot] = [
                    pltpu.async_copy(
                        mbuf.at[slot, q], acc.at[ibuf.at[slot, q]], sems, add=True
                    )
                    for q in range(4)
                ]
            for slot in range(2):
                for d in sd[slot]:
                    d.wait()
            return 0

        lax.fori_loop(0, n_blocks, block, 0)
        plsc.subcore_barrier()
        pltpu.sync_copy(acc.at[pl.ds(s * npw, npw)], out_hbm.at[c, pl.ds(s * npw, npw)])

    return pl.kernel(
        body,
        out_type=jax.ShapeDtypeStruct((NC, NF), jnp.float32),
        mesh=_sc_mesh(),
        scratch_types=[
            pltpu.VMEM((BK, 128), jnp.int32),
            pltpu.VMEM((BK, 128), jnp.int32),
            pltpu.VMEM((BK, 128), jnp.float32),
            pltpu.VMEM((2, 32, 128), jnp.float32),
            pltpu.VMEM((2, 4, 128), jnp.float32),
            pltpu.VMEM((2, 4, 128), jnp.int32),
            pltpu.VMEM((ZR,), jnp.float32),
            pltpu.MemorySpace.VMEM_SHARED((NF,), jnp.float32),
            pltpu.SemaphoreType.DMA,
            pltpu.SemaphoreType.DMA,
        ],
    )(row2d, col2d, w2d, tab)


def _dis_call(degp3):
    NR = degp3.shape[1]
    BR = 784

    def body(d_ref, o_ref):
        deg = d_ref[0] + d_ref[1] + 1.0
        o_ref[...] = jnp.where(deg > 0, lax.rsqrt(deg), 0.0)

    return pl.pallas_call(
        body,
        grid=(NR // BR,),
        in_specs=[pl.BlockSpec((2, BR, 16), lambda i: (0, i, 0))],
        out_specs=pl.BlockSpec((BR, 16), lambda i: (i, 0)),
        out_shape=jax.ShapeDtypeStruct((NR, 16), jnp.float32),
    )(degp3)


def _dense_call(x, Wf, bf, Wc, dis16):
    NN, F = x.shape
    BR = 800

    def body(x_ref, wf_ref, bf_ref, wc_ref, d_ref, o_ref, o128_ref):
        h = lax.dot_general(
            x_ref[...], wf_ref[...], (((1,), (1,)), ((), ())),
            preferred_element_type=jnp.float32,
        )
        h = jnp.maximum(h + bf_ref[...], 0.0)
        h2 = lax.dot_general(
            h, wc_ref[...], (((1,), (1,)), ((), ())),
            preferred_element_type=jnp.float32,
        )
        t = h2 * d_ref[...]
        o_ref[...] = t
        o128_ref[...] = jnp.concatenate([t] * 8, axis=1)

    return pl.pallas_call(
        body,
        grid=(NN // BR,),
        in_specs=[
            pl.BlockSpec((BR, F), lambda i: (i, 0)),
            pl.BlockSpec((HID, F), lambda i: (0, 0)),
            pl.BlockSpec((1, HID), lambda i: (0, 0)),
            pl.BlockSpec((HID, HID), lambda i: (0, 0)),
            pl.BlockSpec((BR, HID), lambda i: (i, 0)),
        ],
        out_specs=[
            pl.BlockSpec((BR, HID), lambda i: (i, 0)),
            pl.BlockSpec((BR, 128), lambda i: (i, 0)),
        ],
        out_shape=[
            jax.ShapeDtypeStruct((NN, HID), jnp.float32),
            jax.ShapeDtypeStruct((NN, 128), jnp.float32),
        ],
    )(x, Wf, bf.reshape(1, HID), Wc, dis16)


def _final_call(accp, h2s, dis16, bc, Wl, bl):
    NN = h2s.shape[0]
    C = Wl.shape[0]
    BR = 800

    def body(a_ref, h_ref, d_ref, bc_ref, wl_ref, bl_ref, o_ref):
        m = a_ref[0] + a_ref[1] + h_ref[...]
        h = jnp.maximum(d_ref[...] * m + bc_ref[...], 0.0)
        logits = lax.dot_general(
            h, wl_ref[...], (((1,), (1,)), ((), ())),
            preferred_element_type=jnp.float32,
        ) + bl_ref[...]
        mx = jnp.max(logits, axis=-1, keepdims=True)
        ex = jnp.exp(logits - mx)
        lse = mx + jnp.log(jnp.sum(ex, axis=-1, keepdims=True))
        o_ref[...] = logits - lse

    return pl.pallas_call(
        body,
        grid=(NN // BR,),
        in_specs=[
            pl.BlockSpec((2, BR, HID), lambda i: (0, i, 0)),
            pl.BlockSpec((BR, HID), lambda i: (i, 0)),
            pl.BlockSpec((BR, HID), lambda i: (i, 0)),
            pl.BlockSpec((1, HID), lambda i: (0, 0)),
            pl.BlockSpec((C, HID), lambda i: (0, 0)),
            pl.BlockSpec((1, C), lambda i: (0, 0)),
        ],
        out_specs=pl.BlockSpec((BR, C), lambda i: (i, 0)),
        out_shape=jax.ShapeDtypeStruct((NN, C), jnp.float32),
    )(accp, h2s, dis16, bc.reshape(1, HID), Wl, bl.reshape(1, C))


def kernel(x, edge_index, edge_weight, W_first, b_first, W_conv, b_conv, W_lin2, b_lin2):
    NN, _ = x.shape
    E = edge_index.shape[1]

    NP = ((NN + 2047) // 2048) * 2048
    base_rows = -(-E // 128)
    RE = -(-base_rows // (NW * 16)) * (NW * 16)
    padE = RE * 128 - E

    row = edge_index[0].astype(jnp.int32)
    col = edge_index[1].astype(jnp.int32)
    w = edge_weight.astype(jnp.float32)
    ar = jnp.arange(padE, dtype=jnp.int32)
    pad_dst = NN + ar % (NP - NN) if NP > NN else ar % NN
    row2d = jnp.concatenate([row, ar % NN]).reshape(RE, 128)
    col2d = jnp.concatenate([col, pad_dst]).reshape(RE, 128)
    w2d = jnp.concatenate([w, jnp.zeros((padE,), jnp.float32)]).reshape(RE, 128)

    degp = _deg_call(col2d, w2d, NP)
    dis2d = _dis_call(degp.reshape(2, NP // 16, 16))
    dis16 = jnp.broadcast_to(dis2d.reshape(NP)[:NN, None], (NN, HID))
    h2s, tab128 = _dense_call(x, W_first, b_first, W_conv, dis16)
    accp = _msg_call(row2d, col2d, w2d, tab128, NP).reshape(2, NP, HID)
    return _final_call(accp, h2s, dis16, b_conv, W_lin2, b_lin2)
